# baseline (speedup 1.0000x reference)
import functools

import jax
import jax.numpy as jnp
from jax import lax
from jax.experimental import pallas as pl
from jax.experimental.pallas import tpu as pltpu
from jax.experimental.pallas import tpu_sc as plsc

BATCH = 16384
DIM = 64
WIDE = 2 * DIM
NC = 2
NS = 16
NW = NC * NS
BPW = BATCH // NW
CHUNK = 128
NCHUNK = BPW // CHUNK
HALF = BPW // 2


def _sc_gather(idx_h, idx_r, idx_t, ent128, rel128):
    mesh = plsc.VectorSubcoreMesh(core_axis_name="c", subcore_axis_name="s")
    row_t = jax.ShapeDtypeStruct((BATCH, WIDE), jnp.float32)

    @functools.partial(
        pl.kernel,
        mesh=mesh,
        out_type=[row_t, row_t, row_t],
        compiler_params=pltpu.CompilerParams(use_tc_tiling_on_sc=False),
        scratch_types=[
            pltpu.VMEM((NCHUNK, CHUNK), jnp.int32),
            pltpu.VMEM((NCHUNK, CHUNK), jnp.int32),
            pltpu.VMEM((NCHUNK, CHUNK), jnp.int32),
            pltpu.VMEM((HALF, WIDE), jnp.float32),
            pltpu.VMEM((HALF, WIDE), jnp.float32),
            pltpu.SemaphoreType.DMA,
            pltpu.SemaphoreType.DMA,
            pltpu.SemaphoreType.DMA,
        ],
    )
    def k(ih_hbm, ir_hbm, it_hbm, ent_hbm, rel_hbm,
          oh_hbm, ot_hbm, orel_hbm,
          ih_v, ir_v, it_v, rows0_v, rows1_v, gsem, ssem0, ssem1):
        wid = lax.axis_index("s") * NC + lax.axis_index("c")
        base = wid * BPW
        crow = wid * NCHUNK
        pltpu.sync_copy(ih_hbm.at[pl.ds(crow, NCHUNK)], ih_v)
        pltpu.sync_copy(ir_hbm.at[pl.ds(crow, NCHUNK)], ir_v)
        pltpu.sync_copy(it_hbm.at[pl.ds(crow, NCHUNK)], it_v)

        rows = (rows0_v, rows1_v)
        ssems = (ssem0, ssem1)
        stages = []
        for idx_v, table, out in (
            (ih_v, ent_hbm, oh_hbm),
            (it_v, ent_hbm, ot_hbm),
            (ir_v, rel_hbm, orel_hbm),
        ):
            for half in range(2):
                stages.append((table, idx_v, out, half))

        def fire_gather(s):
            table, idx_v, _, half = stages[s]
            buf = rows[s % 2]
            return [
                pltpu.async_copy(
                    table.at[idx_v.at[2 * half + c]],
                    buf.at[pl.ds(c * CHUNK, CHUNK)],
                    gsem,
                )
                for c in range(2)
            ]

        def fire_store(s):
            out, half = stages[s][2], stages[s][3]
            buf = rows[s % 2]
            return pltpu.async_copy(
                buf,
                out.at[pl.ds(base + half * HALF, HALF)],
                ssems[s % 2],
            )

        stores = [None] * 6
        g = fire_gather(0)
        for s in range(6):
            for hnd in g:
                hnd.wait()
            stores[s] = fire_store(s)
            if s < 5:
                if s >= 1:
                    stores[s - 1].wait()
                g = fire_gather(s + 1)
        stores[4].wait()
        stores[5].wait()

    return k(idx_h, idx_r, idx_t, ent128, rel128)


def _score_body(h_ref, t_ref, rel_ref, o_ref):
    h = h_ref[...][:, :DIM]
    t = t_ref[...][:, :DIM]
    rel = rel_ref[...]
    dr = rel[:, :DIM]
    w = rel[:, DIM:]
    nw = jnp.sqrt(jnp.sum(w * w, axis=-1, keepdims=True))
    wn = w / jnp.maximum(nw, 1e-12)
    hv = h - jnp.sum(h * wn, axis=-1, keepdims=True) * wn
    tv = t - jnp.sum(t * wn, axis=-1, keepdims=True) * wn
    hn = jnp.sqrt(jnp.sum(hv * hv, axis=-1, keepdims=True))
    hv = hv / jnp.maximum(hn, 1e-12)
    tn = jnp.sqrt(jnp.sum(tv * tv, axis=-1, keepdims=True))
    tv = tv / jnp.maximum(tn, 1e-12)
    diff = hv + dr - tv
    o_ref[...] = jnp.sqrt(jnp.sum(diff * diff, axis=-1))


def _tc_score(h, t, rel):
    bt = 2048
    grid = (BATCH // bt,)
    spec = pl.BlockSpec((bt, WIDE), lambda i: (i, 0))
    return pl.pallas_call(
        _score_body,
        grid=grid,
        in_specs=[spec, spec, spec],
        out_specs=pl.BlockSpec((bt,), lambda i: (i,)),
        out_shape=jax.ShapeDtypeStruct((BATCH,), jnp.float32),
    )(h, t, rel)


def kernel(sample, entity_embedding, translation_embedding, norm_vector):
    sample = sample.astype(jnp.int32)
    idx_h = sample[:, 0].reshape(NW * NCHUNK, CHUNK)
    idx_r = sample[:, 1].reshape(NW * NCHUNK, CHUNK)
    idx_t = sample[:, 2].reshape(NW * NCHUNK, CHUNK)
    ent128 = jnp.concatenate(
        [entity_embedding, translation_embedding], axis=1)
    rel128 = jnp.concatenate([translation_embedding, norm_vector], axis=1)
    h, t, rel = _sc_gather(idx_h, idx_r, idx_t, ent128, rel128)
    return _tc_score(h, t, rel)


# speedup vs baseline: 1.0968x; 1.0968x over previous
import functools

import jax
import jax.numpy as jnp
from jax import lax
from jax.experimental import pallas as pl
from jax.experimental.pallas import tpu as pltpu
from jax.experimental.pallas import tpu_sc as plsc

BATCH = 16384
DIM = 64
WIDE = 2 * DIM
NC = 2
NS = 16
NW = NC * NS
BPW = BATCH // NW
CHUNK = 128
NCHUNK = BPW // CHUNK
HALF = BPW // 2


def _sc_gather(idx_h, idx_r, idx_t, ent128, rel128):
    mesh = plsc.VectorSubcoreMesh(core_axis_name="c", subcore_axis_name="s")
    row_t = jax.ShapeDtypeStruct((BATCH, WIDE), jnp.float32)

    @functools.partial(
        pl.kernel,
        mesh=mesh,
        out_type=[row_t, row_t, row_t],
        compiler_params=pltpu.CompilerParams(use_tc_tiling_on_sc=False),
        scratch_types=[
            pltpu.VMEM((NCHUNK, CHUNK), jnp.int32),
            pltpu.VMEM((NCHUNK, CHUNK), jnp.int32),
            pltpu.VMEM((NCHUNK, CHUNK), jnp.int32),
            pltpu.VMEM((HALF, WIDE), jnp.float32),
            pltpu.VMEM((HALF, WIDE), jnp.float32),
            pltpu.SemaphoreType.DMA,
            pltpu.SemaphoreType.DMA,
            pltpu.SemaphoreType.DMA,
        ],
    )
    def k(ih_hbm, ir_hbm, it_hbm, ent_hbm, rel_hbm,
          oh_hbm, ot_hbm, orel_hbm,
          ih_v, ir_v, it_v, rows0_v, rows1_v, gsem, ssem0, ssem1):
        wid = lax.axis_index("s") * NC + lax.axis_index("c")
        base = wid * BPW
        crow = wid * NCHUNK
        pltpu.sync_copy(ih_hbm.at[pl.ds(crow, NCHUNK)], ih_v)
        pltpu.sync_copy(ir_hbm.at[pl.ds(crow, NCHUNK)], ir_v)
        pltpu.sync_copy(it_hbm.at[pl.ds(crow, NCHUNK)], it_v)

        rows = (rows0_v, rows1_v)
        ssems = (ssem0, ssem1)
        stages = []
        for idx_v, table, out in (
            (ih_v, ent_hbm, oh_hbm),
            (it_v, ent_hbm, ot_hbm),
            (ir_v, rel_hbm, orel_hbm),
        ):
            for half in range(2):
                stages.append((table, idx_v, out, half))

        def fire_gather(s):
            table, idx_v, _, half = stages[s]
            buf = rows[s % 2]
            return [
                pltpu.async_copy(
                    table.at[idx_v.at[2 * half + c]],
                    buf.at[pl.ds(c * CHUNK, CHUNK)],
                    gsem,
                )
                for c in range(2)
            ]

        def fire_store(s):
            out, half = stages[s][2], stages[s][3]
            buf = rows[s % 2]
            return pltpu.async_copy(
                buf,
                out.at[pl.ds(base + half * HALF, HALF)],
                ssems[s % 2],
            )

        stores = [None] * 6
        g = fire_gather(0)
        for s in range(6):
            for hnd in g:
                hnd.wait()
            stores[s] = fire_store(s)
            if s < 5:
                if s >= 1:
                    stores[s - 1].wait()
                g = fire_gather(s + 1)
        stores[4].wait()
        stores[5].wait()

    return k(idx_h, idx_r, idx_t, ent128, rel128)


def _score_body(h_ref, t_ref, rel_ref, o_ref):
    h = h_ref[...][:, :DIM]
    t = t_ref[...][:, :DIM]
    rel = rel_ref[...]
    dr = rel[:, :DIM]
    w = rel[:, DIM:]
    nw = jnp.sqrt(jnp.sum(w * w, axis=-1, keepdims=True))
    wn = w / jnp.maximum(nw, 1e-12)
    hv = h - jnp.sum(h * wn, axis=-1, keepdims=True) * wn
    tv = t - jnp.sum(t * wn, axis=-1, keepdims=True) * wn
    hn = jnp.sqrt(jnp.sum(hv * hv, axis=-1, keepdims=True))
    hv = hv / jnp.maximum(hn, 1e-12)
    tn = jnp.sqrt(jnp.sum(tv * tv, axis=-1, keepdims=True))
    tv = tv / jnp.maximum(tn, 1e-12)
    diff = hv + dr - tv
    o_ref[...] = jnp.sqrt(jnp.sum(diff * diff, axis=-1))


def _tc_score(h, t, rel):
    bt = 2048
    grid = (BATCH // bt,)
    spec = pl.BlockSpec((bt, WIDE), lambda i: (i, 0))
    return pl.pallas_call(
        _score_body,
        grid=grid,
        in_specs=[spec, spec, spec],
        out_specs=pl.BlockSpec((bt,), lambda i: (i,)),
        out_shape=jax.ShapeDtypeStruct((BATCH,), jnp.float32),
    )(h, t, rel)


def kernel(sample, entity_embedding, translation_embedding, norm_vector):
    sample = sample.astype(jnp.int32)
    idx_h = sample[:, 0].reshape(NW * NCHUNK, CHUNK)
    idx_r = sample[:, 1].reshape(NW * NCHUNK, CHUNK)
    idx_t = sample[:, 2].reshape(NW * NCHUNK, CHUNK)
    ent128 = jnp.pad(entity_embedding, ((0, 0), (0, DIM)))
    rel128 = jnp.concatenate([translation_embedding, norm_vector], axis=1)
    h, t, rel = _sc_gather(idx_h, idx_r, idx_t, ent128, rel128)
    return _tc_score(h, t, rel)
